# Initial kernel scaffold; baseline (speedup 1.0000x reference)
import jax
import jax.numpy as jnp
from jax.experimental import pallas as pl


def kernel(vertices, dir0, w1, b1, d1, w3, b3, d3, w4, b4, d4, w6, b6, d6, w7, b7, d7, w8, b8, d8, w9, b9, d9, w10, b10, d10, w11, b11, d11, cw1, cb1, bng, bnb, cw2, cb2):
    raise NotImplementedError("write your pallas kernel here")


# speedup vs baseline: 1.1441x; 1.1441x over previous
import functools

import jax
import jax.numpy as jnp
from jax.experimental import pallas as pl

NSAMPLE = 32


def _ball_body(pi_ref, pj_ref, o_ref, *, radius, n, nsample):
    pi = pi_ref[0]
    pj = pj_ref[0]
    ssi = jnp.sum(pi * pi, axis=1, keepdims=True)
    ssj = jnp.sum(pj * pj, axis=0, keepdims=True)
    dot = jnp.dot(pi, pj, preferred_element_type=jnp.float32)
    d2 = (-2.0 * dot + ssi) + ssj
    r = pi.shape[0]
    iota = jax.lax.broadcasted_iota(jnp.int32, (r, n), 1)
    cand = jnp.where(d2 <= radius * radius, iota, n)
    cols = []
    first = None
    for _ in range(nsample):
        m = jnp.min(cand, axis=1, keepdims=True)
        if first is None:
            first = m
        cols.append(jnp.where(m == n, first, m))
        cand = jnp.where(cand == m, n, cand)
    o_ref[0] = jnp.concatenate(cols, axis=1)


def _ball_query(vertices, radius):
    b, n, _ = vertices.shape
    rows = min(n, 256)
    vt = jnp.transpose(vertices, (0, 2, 1))
    return pl.pallas_call(
        functools.partial(_ball_body, radius=radius, n=n, nsample=NSAMPLE),
        grid=(b, n // rows),
        in_specs=[pl.BlockSpec((1, rows, 3), lambda bb, ii: (bb, ii, 0)),
                  pl.BlockSpec((1, 3, n), lambda bb, ii: (bb, 0, 0))],
        out_specs=pl.BlockSpec((1, rows, NSAMPLE), lambda bb, ii: (bb, ii, 0)),
        out_shape=jax.ShapeDtypeStruct((b, n, NSAMPLE), jnp.int32),
    )(vertices, vt)


def _ref_qbp(radius, nsample, xyz):
    b, n, _ = xyz.shape
    d = -2.0 * jnp.einsum('bnc,bmc->bnm', xyz, xyz)
    d = d + jnp.sum(xyz * xyz, axis=-1)[:, :, None]
    d = d + jnp.sum(xyz * xyz, axis=-1)[:, None, :]
    gi = jnp.broadcast_to(jnp.arange(n, dtype=jnp.int32)[None, None, :], (b, n, n))
    gi = jnp.where(d > radius * radius, n, gi)
    gi = jnp.sort(gi, axis=-1)[:, :, :nsample]
    gf = jnp.broadcast_to(gi[:, :, :1], gi.shape)
    return jnp.where(gi == n, gf, gi)


def _normalize(x, axis):
    nrm = jnp.linalg.norm(x, axis=axis, keepdims=True)
    return x / jnp.maximum(nrm, 1e-12)


def _gather_rows(tensor, index):
    id0 = jnp.arange(tensor.shape[0])[:, None, None]
    return tensor[id0, index]


def _neighbor_dir_norm(vertices, ni):
    neighbors = _gather_rows(vertices, ni)
    direction = neighbors - vertices[:, :, None, :]
    return _normalize(direction, axis=-1)


def _conv_surface(ni, vertices, directions, kernel_num):
    bs, v, n = ni.shape
    nd = _neighbor_dir_norm(vertices, ni)
    sd = _normalize(directions, axis=0)
    theta = jax.nn.relu(nd @ sd)
    theta = theta.reshape(bs, v, n, 1, kernel_num)
    return jnp.sum(jnp.max(theta, axis=2), axis=2)


def _conv_layer(ni, vertices, fm, w, bias, directions, out_channel):
    bs, v, n = ni.shape
    nd = _neighbor_dir_norm(vertices, ni)
    sd = _normalize(directions, axis=0)
    theta = nd @ sd
    feature_out = fm @ w + bias
    fc = feature_out[:, :, :out_channel]
    fs = _gather_rows(feature_out[:, :, out_channel:], ni)
    act = (theta * fs).reshape(bs, v, n, 1, out_channel)
    act = jnp.sum(jnp.max(act, axis=2), axis=2)
    return fc + act


def _pool(vertices, fm, ni):
    v = vertices.shape[1]
    pool_num = v // 2
    ni4 = ni[:, :pool_num, :4]
    pooled = jnp.max(_gather_rows(fm, ni4), axis=2)
    return vertices[:, :pool_num, :], pooled


def kernel(vertices, dir0, w1, b1, d1, w3, b3, d3, w4, b4, d4, w6, b6, d6,
           w7, b7, d7, w8, b8, d8, w9, b9, d9, w10, b10, d10, w11, b11, d11,
           cw1, cb1, bng, bnb, cw2, cb2):
    relu = jax.nn.relu
    ni = _ball_query(vertices, 0.2)
    fm0 = relu(_conv_surface(ni, vertices, dir0, 32))
    fm1 = relu(_conv_layer(ni, vertices, fm0, w1, b1, d1, 32))
    fm1 = jnp.concatenate([fm0, fm1], axis=2)
    vertices, fm1 = _pool(vertices, fm1, ni)

    ni = _ball_query(vertices, 0.4)
    fm3 = relu(_conv_layer(ni, vertices, fm1, w3, b3, d3, 32))
    fm3 = jnp.concatenate([fm1, fm3], axis=2)
    fm4 = relu(_conv_layer(ni, vertices, fm3, w4, b4, d4, 32))
    fm4 = jnp.concatenate([fm3, fm4], axis=2)
    vertices, fm4 = _pool(vertices, fm4, ni)

    ni = _ball_query(vertices, 0.6)
    fm6 = relu(_conv_layer(ni, vertices, fm4, w6, b6, d6, 32))
    fm6 = jnp.concatenate([fm4, fm6], axis=2)
    fm7 = relu(_conv_layer(ni, vertices, fm6, w7, b7, d7, 32))
    fm7 = jnp.concatenate([fm6, fm7], axis=2)
    vertices, fm7 = _pool(vertices, fm7, ni)

    ni = _ball_query(vertices, 0.8)
    fm8 = relu(_conv_layer(ni, vertices, fm7, w8, b8, d8, 32))
    fm8 = jnp.concatenate([fm7, fm8], axis=2)
    fm9 = relu(_conv_layer(ni, vertices, fm8, w9, b9, d9, 32))
    fm9 = jnp.concatenate([fm8, fm9], axis=2)
    vertices, fm9 = _pool(vertices, fm9, ni)

    ni = _ball_query(vertices, 1.0)
    fm10 = relu(_conv_layer(ni, vertices, fm9, w10, b10, d10, 32))
    fm10 = jnp.concatenate([fm9, fm10], axis=2)
    fm11 = _conv_layer(ni, vertices, fm10, w11, b11, d11, 1024)
    feature_global = jnp.max(fm11, axis=1)
    x = feature_global @ cw1 + cb1
    x = bng * x / jnp.sqrt(1.0 + 1e-5) + bnb
    x = relu(x)
    return x @ cw2 + cb2


# speedup vs baseline: 2.1218x; 1.8545x over previous
import functools

import jax
import jax.numpy as jnp
from jax.experimental import pallas as pl

NSAMPLE = 32
_HI = jax.lax.Precision.HIGHEST


def _ball_body(pi_ref, pj_ref, o_ref, *, radius, n, nsample):
    pi = pi_ref[0]
    pj = pj_ref[0]
    ssi = jnp.sum(pi * pi, axis=1, keepdims=True)
    ssj = jnp.sum(pj * pj, axis=0, keepdims=True)
    dot = jnp.dot(pi, pj, preferred_element_type=jnp.float32)
    d2 = (-2.0 * dot + ssi) + ssj
    r = pi.shape[0]
    iota = jax.lax.broadcasted_iota(jnp.int32, (r, n), 1)
    cand = jnp.where(d2 <= radius * radius, iota, n)
    cols = []
    first = None
    for _ in range(nsample):
        m = jnp.min(cand, axis=1, keepdims=True)
        if first is None:
            first = m
        cols.append(jnp.where(m == n, first, m))
        cand = jnp.where(cand == m, n, cand)
    o_ref[0] = jnp.concatenate(cols, axis=1)


def _ball_query(vertices, radius):
    b, n, _ = vertices.shape
    rows = min(n, 256)
    vt = jnp.transpose(vertices, (0, 2, 1))
    return pl.pallas_call(
        functools.partial(_ball_body, radius=radius, n=n, nsample=NSAMPLE),
        grid=(b, n // rows),
        in_specs=[pl.BlockSpec((1, rows, 3), lambda bb, ii: (bb, ii, 0)),
                  pl.BlockSpec((1, 3, n), lambda bb, ii: (bb, 0, 0))],
        out_specs=pl.BlockSpec((1, rows, NSAMPLE), lambda bb, ii: (bb, ii, 0)),
        out_shape=jax.ShapeDtypeStruct((b, n, NSAMPLE), jnp.int32),
    )(vertices, vt)


def _mm_body(fm_ref, w_ref, b_ref, o_ref):
    o_ref[0] = (jnp.dot(fm_ref[0], w_ref[...],
                        preferred_element_type=jnp.float32) + b_ref[...])


def _mm(fm, w, bias):
    b, n, ci = fm.shape
    c2 = w.shape[1]
    return pl.pallas_call(
        _mm_body,
        grid=(b,),
        in_specs=[pl.BlockSpec((1, n, ci), lambda bb: (bb, 0, 0)),
                  pl.BlockSpec((ci, c2), lambda bb: (0, 0)),
                  pl.BlockSpec((1, c2), lambda bb: (0, 0))],
        out_specs=pl.BlockSpec((1, n, c2), lambda bb: (bb, 0, 0)),
        out_shape=jax.ShapeDtypeStruct((b, n, c2), jnp.float32),
    )(fm, w, bias.reshape(1, -1))


def _conv_body(ni_ref, xyzb_ref, xyzf_ref, foutb_ref, foutf_ref, sd_ref,
               o_ref, *, co, n, nsample):
    rows = ni_ref.shape[1]
    xyz_b = xyzb_ref[0]
    xyz_f = xyzf_ref[0]
    fout_f = foutf_ref[0]
    fc = foutb_ref[0][:, :co]
    sd = sd_ref[...]
    sdn = sd / jnp.maximum(
        jnp.sqrt(jnp.sum(sd * sd, axis=0, keepdims=True)), 1e-12)
    cat = jnp.concatenate([xyz_f, fout_f[:, co:]], axis=1)
    iota = jax.lax.broadcasted_iota(jnp.int32, (rows, n), 1)
    acc = jnp.full((rows, co), -jnp.inf, jnp.float32)
    for k in range(nsample):
        idx = ni_ref[0, :, k:k + 1]
        oh = jnp.where(iota == idx, 1.0, 0.0)
        g = jax.lax.dot(oh, cat, precision=_HI,
                        preferred_element_type=jnp.float32)
        d = g[:, :3] - xyz_b
        nrm = jnp.sqrt(jnp.sum(d * d, axis=1, keepdims=True))
        dn = d / jnp.maximum(nrm, 1e-12)
        theta = jnp.dot(dn, sdn, preferred_element_type=jnp.float32)
        acc = jnp.maximum(acc, theta * g[:, 3:])
    o_ref[0] = fc + acc


def _conv(ni, vertices, fout, sd, co):
    b, n, _ = ni.shape
    rows = min(n, 256)
    c2 = fout.shape[2]
    return pl.pallas_call(
        functools.partial(_conv_body, co=co, n=n, nsample=NSAMPLE),
        grid=(b, n // rows),
        in_specs=[pl.BlockSpec((1, rows, NSAMPLE), lambda bb, ii: (bb, ii, 0)),
                  pl.BlockSpec((1, rows, 3), lambda bb, ii: (bb, ii, 0)),
                  pl.BlockSpec((1, n, 3), lambda bb, ii: (bb, 0, 0)),
                  pl.BlockSpec((1, rows, c2), lambda bb, ii: (bb, ii, 0)),
                  pl.BlockSpec((1, n, c2), lambda bb, ii: (bb, 0, 0)),
                  pl.BlockSpec((3, co), lambda bb, ii: (0, 0))],
        out_specs=pl.BlockSpec((1, rows, co), lambda bb, ii: (bb, ii, 0)),
        out_shape=jax.ShapeDtypeStruct((b, n, co), jnp.float32),
    )(ni, vertices, vertices, fout, fout, sd)


def _conv_layer(ni, vertices, fm, w, bias, directions, co):
    fout = _mm(fm, w, bias)
    return _conv(ni, vertices, fout, directions, co)


def _surf_body(ni_ref, xyzb_ref, xyzf_ref, sd_ref, o_ref, *, co, n, nsample):
    rows = ni_ref.shape[1]
    xyz_b = xyzb_ref[0]
    xyz_f = xyzf_ref[0]
    sd = sd_ref[...]
    sdn = sd / jnp.maximum(
        jnp.sqrt(jnp.sum(sd * sd, axis=0, keepdims=True)), 1e-12)
    iota = jax.lax.broadcasted_iota(jnp.int32, (rows, n), 1)
    acc = jnp.full((rows, co), -jnp.inf, jnp.float32)
    for k in range(nsample):
        idx = ni_ref[0, :, k:k + 1]
        oh = jnp.where(iota == idx, 1.0, 0.0)
        g = jax.lax.dot(oh, xyz_f, precision=_HI,
                        preferred_element_type=jnp.float32)
        d = g - xyz_b
        nrm = jnp.sqrt(jnp.sum(d * d, axis=1, keepdims=True))
        dn = d / jnp.maximum(nrm, 1e-12)
        theta = jnp.dot(dn, sdn, preferred_element_type=jnp.float32)
        acc = jnp.maximum(acc, jnp.maximum(theta, 0.0))
    o_ref[0] = acc


def _conv_surface(ni, vertices, directions, co):
    b, n, _ = ni.shape
    rows = min(n, 256)
    return pl.pallas_call(
        functools.partial(_surf_body, co=co, n=n, nsample=NSAMPLE),
        grid=(b, n // rows),
        in_specs=[pl.BlockSpec((1, rows, NSAMPLE), lambda bb, ii: (bb, ii, 0)),
                  pl.BlockSpec((1, rows, 3), lambda bb, ii: (bb, ii, 0)),
                  pl.BlockSpec((1, n, 3), lambda bb, ii: (bb, 0, 0)),
                  pl.BlockSpec((3, co), lambda bb, ii: (0, 0))],
        out_specs=pl.BlockSpec((1, rows, co), lambda bb, ii: (bb, ii, 0)),
        out_shape=jax.ShapeDtypeStruct((b, n, co), jnp.float32),
    )(ni, vertices, vertices, directions)


def _pool_body(ni_ref, fmf_ref, o_ref, *, n):
    rows = ni_ref.shape[1]
    fm = fmf_ref[0]
    iota = jax.lax.broadcasted_iota(jnp.int32, (rows, n), 1)
    acc = jnp.full((rows, fm.shape[1]), -jnp.inf, jnp.float32)
    for k in range(4):
        idx = ni_ref[0, :, k:k + 1]
        oh = jnp.where(iota == idx, 1.0, 0.0)
        g = jax.lax.dot(oh, fm, precision=_HI,
                        preferred_element_type=jnp.float32)
        acc = jnp.maximum(acc, g)
    o_ref[0] = acc


def _pool(vertices, fm, ni):
    b, n, c = fm.shape
    pool_num = n // 2
    rows = min(pool_num, 256)
    ni4 = ni[:, :pool_num, :4]
    pooled = pl.pallas_call(
        functools.partial(_pool_body, n=n),
        grid=(b, pool_num // rows),
        in_specs=[pl.BlockSpec((1, rows, 4), lambda bb, ii: (bb, ii, 0)),
                  pl.BlockSpec((1, n, c), lambda bb, ii: (bb, 0, 0))],
        out_specs=pl.BlockSpec((1, rows, c), lambda bb, ii: (bb, ii, 0)),
        out_shape=jax.ShapeDtypeStruct((b, pool_num, c), jnp.float32),
    )(ni4, fm)
    return vertices[:, :pool_num, :], pooled


def _tail_body(fm_ref, cw1_ref, cb1_ref, bng_ref, bnb_ref, cw2_ref, cb2_ref,
               o_ref):
    m = jnp.max(fm_ref[0], axis=0, keepdims=True)
    x = jnp.dot(m, cw1_ref[...], preferred_element_type=jnp.float32)
    x = x + cb1_ref[...]
    x = bng_ref[...] * x / jnp.sqrt(1.0 + 1e-5) + bnb_ref[...]
    x = jnp.maximum(x, 0.0)
    o_ref[0] = (jnp.dot(x, cw2_ref[...], preferred_element_type=jnp.float32)
                + cb2_ref[...])


def _tail(fm11, cw1, cb1, bng, bnb, cw2, cb2):
    b, n, c = fm11.shape
    h = cw1.shape[1]
    o = cw2.shape[1]
    out = pl.pallas_call(
        _tail_body,
        grid=(b,),
        in_specs=[pl.BlockSpec((1, n, c), lambda bb: (bb, 0, 0)),
                  pl.BlockSpec((c, h), lambda bb: (0, 0)),
                  pl.BlockSpec((1, h), lambda bb: (0, 0)),
                  pl.BlockSpec((1, h), lambda bb: (0, 0)),
                  pl.BlockSpec((1, h), lambda bb: (0, 0)),
                  pl.BlockSpec((h, o), lambda bb: (0, 0)),
                  pl.BlockSpec((1, o), lambda bb: (0, 0))],
        out_specs=pl.BlockSpec((1, 1, o), lambda bb: (bb, 0, 0)),
        out_shape=jax.ShapeDtypeStruct((b, 1, o), jnp.float32),
    )(fm11, cw1, cb1.reshape(1, -1), bng.reshape(1, -1), bnb.reshape(1, -1),
      cw2, cb2.reshape(1, -1))
    return out.reshape(b, o)


def kernel(vertices, dir0, w1, b1, d1, w3, b3, d3, w4, b4, d4, w6, b6, d6,
           w7, b7, d7, w8, b8, d8, w9, b9, d9, w10, b10, d10, w11, b11, d11,
           cw1, cb1, bng, bnb, cw2, cb2):
    relu = jax.nn.relu
    ni = _ball_query(vertices, 0.2)
    fm0 = relu(_conv_surface(ni, vertices, dir0, 32))
    fm1 = relu(_conv_layer(ni, vertices, fm0, w1, b1, d1, 32))
    fm1 = jnp.concatenate([fm0, fm1], axis=2)
    vertices, fm1 = _pool(vertices, fm1, ni)

    ni = _ball_query(vertices, 0.4)
    fm3 = relu(_conv_layer(ni, vertices, fm1, w3, b3, d3, 32))
    fm3 = jnp.concatenate([fm1, fm3], axis=2)
    fm4 = relu(_conv_layer(ni, vertices, fm3, w4, b4, d4, 32))
    fm4 = jnp.concatenate([fm3, fm4], axis=2)
    vertices, fm4 = _pool(vertices, fm4, ni)

    ni = _ball_query(vertices, 0.6)
    fm6 = relu(_conv_layer(ni, vertices, fm4, w6, b6, d6, 32))
    fm6 = jnp.concatenate([fm4, fm6], axis=2)
    fm7 = relu(_conv_layer(ni, vertices, fm6, w7, b7, d7, 32))
    fm7 = jnp.concatenate([fm6, fm7], axis=2)
    vertices, fm7 = _pool(vertices, fm7, ni)

    ni = _ball_query(vertices, 0.8)
    fm8 = relu(_conv_layer(ni, vertices, fm7, w8, b8, d8, 32))
    fm8 = jnp.concatenate([fm7, fm8], axis=2)
    fm9 = relu(_conv_layer(ni, vertices, fm8, w9, b9, d9, 32))
    fm9 = jnp.concatenate([fm8, fm9], axis=2)
    vertices, fm9 = _pool(vertices, fm9, ni)

    ni = _ball_query(vertices, 1.0)
    fm10 = relu(_conv_layer(ni, vertices, fm9, w10, b10, d10, 32))
    fm10 = jnp.concatenate([fm9, fm10], axis=2)
    fm11 = _conv_layer(ni, vertices, fm10, w11, b11, d11, 1024)
    return _tail(fm11, cw1, cb1, bng, bnb, cw2, cb2)


# speedup vs baseline: 3.9276x; 1.8511x over previous
import functools

import jax
import jax.numpy as jnp
from jax.experimental import pallas as pl

NSAMPLE = 32
_HI = jax.lax.Precision.HIGHEST


def _ball_body(pi_ref, pj_ref, o_ref, *, radius, n, nsample):
    pi = pi_ref[0]
    pj = pj_ref[0]
    ssi = jnp.sum(pi * pi, axis=1, keepdims=True)
    ssj = jnp.sum(pj * pj, axis=0, keepdims=True)
    dot = jnp.dot(pi, pj, preferred_element_type=jnp.float32)
    d2 = (-2.0 * dot + ssi) + ssj
    r = pi.shape[0]
    iota = jax.lax.broadcasted_iota(jnp.int32, (r, n), 1)
    cand = jnp.where(d2 <= radius * radius, iota, n)
    cols = []
    first = None
    for _ in range(nsample):
        m = jnp.min(cand, axis=1, keepdims=True)
        if first is None:
            first = m
        cols.append(jnp.where(m == n, first, m))
        cand = jnp.where(cand == m, n, cand)
    o_ref[0] = jnp.concatenate(cols, axis=1)


def _ball_query(vertices, radius):
    b, n, _ = vertices.shape
    rows = min(n, 256)
    vt = jnp.transpose(vertices, (0, 2, 1))
    return pl.pallas_call(
        functools.partial(_ball_body, radius=radius, n=n, nsample=NSAMPLE),
        grid=(b, n // rows),
        in_specs=[pl.BlockSpec((1, rows, 3), lambda bb, ii: (bb, ii, 0)),
                  pl.BlockSpec((1, 3, n), lambda bb, ii: (bb, 0, 0))],
        out_specs=pl.BlockSpec((1, rows, NSAMPLE), lambda bb, ii: (bb, ii, 0)),
        out_shape=jax.ShapeDtypeStruct((b, n, NSAMPLE), jnp.int32),
    )(vertices, vt)


def _mm_body(fm_ref, w_ref, b_ref, o_ref):
    o_ref[0] = (jnp.dot(fm_ref[0], w_ref[...],
                        preferred_element_type=jnp.float32) + b_ref[...])


def _mm(fm, w, bias):
    b, n, ci = fm.shape
    c2 = w.shape[1]
    return pl.pallas_call(
        _mm_body,
        grid=(b,),
        in_specs=[pl.BlockSpec((1, n, ci), lambda bb: (bb, 0, 0)),
                  pl.BlockSpec((ci, c2), lambda bb: (0, 0)),
                  pl.BlockSpec((1, c2), lambda bb: (0, 0))],
        out_specs=pl.BlockSpec((1, n, c2), lambda bb: (bb, 0, 0)),
        out_shape=jax.ShapeDtypeStruct((b, n, c2), jnp.float32),
    )(fm, w, bias.reshape(1, -1))


def _conv_body(ni_ref, xyzb_ref, xyzf_ref, foutb_ref, foutf_ref, sd_ref,
               o_ref, *, co, n, nsample):
    rows = ni_ref.shape[1]
    xyz_b = xyzb_ref[0]
    xyz_f = xyzf_ref[0]
    fout_f = foutf_ref[0]
    fc = foutb_ref[0][:, :co]
    sd = sd_ref[...]
    sdn = sd / jnp.maximum(
        jnp.sqrt(jnp.sum(sd * sd, axis=0, keepdims=True)), 1e-12)
    cat = jnp.concatenate([xyz_f, fout_f[:, co:]], axis=1)
    iota = jax.lax.broadcasted_iota(jnp.int32, (rows, n), 1)
    dns = []
    fss = []
    for k in range(nsample):
        idx = ni_ref[0, :, k:k + 1]
        oh = jnp.where(iota == idx, 1.0, 0.0)
        g = jax.lax.dot(oh, cat, precision=_HI,
                        preferred_element_type=jnp.float32)
        d = g[:, :3] - xyz_b
        nrm = jnp.sqrt(jnp.sum(d * d, axis=1, keepdims=True))
        dns.append(d / jnp.maximum(nrm, 1e-12))
        fss.append(g[:, 3:])
    dn_all = jnp.concatenate(dns, axis=0)
    theta_all = jnp.dot(dn_all, sdn, preferred_element_type=jnp.float32)
    acc = jnp.full((rows, co), -jnp.inf, jnp.float32)
    for k in range(nsample):
        acc = jnp.maximum(acc, theta_all[k * rows:(k + 1) * rows] * fss[k])
    o_ref[0] = fc + acc


def _conv(ni, vertices, fout, sd, co):
    b, n, _ = ni.shape
    rows = min(n, 256)
    c2 = fout.shape[2]
    return pl.pallas_call(
        functools.partial(_conv_body, co=co, n=n, nsample=NSAMPLE),
        grid=(b, n // rows),
        in_specs=[pl.BlockSpec((1, rows, NSAMPLE), lambda bb, ii: (bb, ii, 0)),
                  pl.BlockSpec((1, rows, 3), lambda bb, ii: (bb, ii, 0)),
                  pl.BlockSpec((1, n, 3), lambda bb, ii: (bb, 0, 0)),
                  pl.BlockSpec((1, rows, c2), lambda bb, ii: (bb, ii, 0)),
                  pl.BlockSpec((1, n, c2), lambda bb, ii: (bb, 0, 0)),
                  pl.BlockSpec((3, co), lambda bb, ii: (0, 0))],
        out_specs=pl.BlockSpec((1, rows, co), lambda bb, ii: (bb, ii, 0)),
        out_shape=jax.ShapeDtypeStruct((b, n, co), jnp.float32),
    )(ni, vertices, vertices, fout, fout, sd)


def _conv_layer(ni, vertices, fm, w, bias, directions, co):
    fout = _mm(fm, w, bias)
    return _conv(ni, vertices, fout, directions, co)


def _surf_body(ni_ref, xyzb_ref, xyzf_ref, sd_ref, o_ref, *, co, n, nsample):
    rows = ni_ref.shape[1]
    xyz_b = xyzb_ref[0]
    xyz_f = xyzf_ref[0]
    sd = sd_ref[...]
    sdn = sd / jnp.maximum(
        jnp.sqrt(jnp.sum(sd * sd, axis=0, keepdims=True)), 1e-12)
    iota = jax.lax.broadcasted_iota(jnp.int32, (rows, n), 1)
    dns = []
    for k in range(nsample):
        idx = ni_ref[0, :, k:k + 1]
        oh = jnp.where(iota == idx, 1.0, 0.0)
        g = jax.lax.dot(oh, xyz_f, precision=_HI,
                        preferred_element_type=jnp.float32)
        d = g - xyz_b
        nrm = jnp.sqrt(jnp.sum(d * d, axis=1, keepdims=True))
        dns.append(d / jnp.maximum(nrm, 1e-12))
    dn_all = jnp.concatenate(dns, axis=0)
    theta_all = jnp.dot(dn_all, sdn, preferred_element_type=jnp.float32)
    acc = jnp.full((rows, co), -jnp.inf, jnp.float32)
    for k in range(nsample):
        acc = jnp.maximum(
            acc, jnp.maximum(theta_all[k * rows:(k + 1) * rows], 0.0))
    o_ref[0] = acc


def _conv_surface(ni, vertices, directions, co):
    b, n, _ = ni.shape
    rows = min(n, 256)
    return pl.pallas_call(
        functools.partial(_surf_body, co=co, n=n, nsample=NSAMPLE),
        grid=(b, n // rows),
        in_specs=[pl.BlockSpec((1, rows, NSAMPLE), lambda bb, ii: (bb, ii, 0)),
                  pl.BlockSpec((1, rows, 3), lambda bb, ii: (bb, ii, 0)),
                  pl.BlockSpec((1, n, 3), lambda bb, ii: (bb, 0, 0)),
                  pl.BlockSpec((3, co), lambda bb, ii: (0, 0))],
        out_specs=pl.BlockSpec((1, rows, co), lambda bb, ii: (bb, ii, 0)),
        out_shape=jax.ShapeDtypeStruct((b, n, co), jnp.float32),
    )(ni, vertices, vertices, directions)


def _pool_body(ni_ref, fmf_ref, o_ref, *, n):
    rows = ni_ref.shape[1]
    fm = fmf_ref[0]
    iota = jax.lax.broadcasted_iota(jnp.int32, (rows, n), 1)
    acc = jnp.full((rows, fm.shape[1]), -jnp.inf, jnp.float32)
    for k in range(4):
        idx = ni_ref[0, :, k:k + 1]
        oh = jnp.where(iota == idx, 1.0, 0.0)
        g = jax.lax.dot(oh, fm, precision=_HI,
                        preferred_element_type=jnp.float32)
        acc = jnp.maximum(acc, g)
    o_ref[0] = acc


def _pool(vertices, fm, ni):
    b, n, c = fm.shape
    pool_num = n // 2
    rows = min(pool_num, 256)
    ni4 = ni[:, :pool_num, :4]
    pooled = pl.pallas_call(
        functools.partial(_pool_body, n=n),
        grid=(b, pool_num // rows),
        in_specs=[pl.BlockSpec((1, rows, 4), lambda bb, ii: (bb, ii, 0)),
                  pl.BlockSpec((1, n, c), lambda bb, ii: (bb, 0, 0))],
        out_specs=pl.BlockSpec((1, rows, c), lambda bb, ii: (bb, ii, 0)),
        out_shape=jax.ShapeDtypeStruct((b, pool_num, c), jnp.float32),
    )(ni4, fm)
    return vertices[:, :pool_num, :], pooled


def _tail_body(fm_ref, cw1_ref, cb1_ref, bng_ref, bnb_ref, cw2_ref, cb2_ref,
               o_ref):
    m = jnp.max(fm_ref[0], axis=0, keepdims=True)
    x = jnp.dot(m, cw1_ref[...], preferred_element_type=jnp.float32)
    x = x + cb1_ref[...]
    x = bng_ref[...] * x / jnp.sqrt(1.0 + 1e-5) + bnb_ref[...]
    x = jnp.maximum(x, 0.0)
    o_ref[0] = (jnp.dot(x, cw2_ref[...], preferred_element_type=jnp.float32)
                + cb2_ref[...])


def _tail(fm11, cw1, cb1, bng, bnb, cw2, cb2):
    b, n, c = fm11.shape
    h = cw1.shape[1]
    o = cw2.shape[1]
    out = pl.pallas_call(
        _tail_body,
        grid=(b,),
        in_specs=[pl.BlockSpec((1, n, c), lambda bb: (bb, 0, 0)),
                  pl.BlockSpec((c, h), lambda bb: (0, 0)),
                  pl.BlockSpec((1, h), lambda bb: (0, 0)),
                  pl.BlockSpec((1, h), lambda bb: (0, 0)),
                  pl.BlockSpec((1, h), lambda bb: (0, 0)),
                  pl.BlockSpec((h, o), lambda bb: (0, 0)),
                  pl.BlockSpec((1, o), lambda bb: (0, 0))],
        out_specs=pl.BlockSpec((1, 1, o), lambda bb: (bb, 0, 0)),
        out_shape=jax.ShapeDtypeStruct((b, 1, o), jnp.float32),
    )(fm11, cw1, cb1.reshape(1, -1), bng.reshape(1, -1), bnb.reshape(1, -1),
      cw2, cb2.reshape(1, -1))
    return out.reshape(b, o)


def kernel(vertices, dir0, w1, b1, d1, w3, b3, d3, w4, b4, d4, w6, b6, d6,
           w7, b7, d7, w8, b8, d8, w9, b9, d9, w10, b10, d10, w11, b11, d11,
           cw1, cb1, bng, bnb, cw2, cb2):
    relu = jax.nn.relu
    ni = _ball_query(vertices, 0.2)
    fm0 = relu(_conv_surface(ni, vertices, dir0, 32))
    fm1 = relu(_conv_layer(ni, vertices, fm0, w1, b1, d1, 32))
    fm1 = jnp.concatenate([fm0, fm1], axis=2)
    vertices, fm1 = _pool(vertices, fm1, ni)

    ni = _ball_query(vertices, 0.4)
    fm3 = relu(_conv_layer(ni, vertices, fm1, w3, b3, d3, 32))
    fm3 = jnp.concatenate([fm1, fm3], axis=2)
    fm4 = relu(_conv_layer(ni, vertices, fm3, w4, b4, d4, 32))
    fm4 = jnp.concatenate([fm3, fm4], axis=2)
    vertices, fm4 = _pool(vertices, fm4, ni)

    ni = _ball_query(vertices, 0.6)
    fm6 = relu(_conv_layer(ni, vertices, fm4, w6, b6, d6, 32))
    fm6 = jnp.concatenate([fm4, fm6], axis=2)
    fm7 = relu(_conv_layer(ni, vertices, fm6, w7, b7, d7, 32))
    fm7 = jnp.concatenate([fm6, fm7], axis=2)
    vertices, fm7 = _pool(vertices, fm7, ni)

    ni = _ball_query(vertices, 0.8)
    fm8 = relu(_conv_layer(ni, vertices, fm7, w8, b8, d8, 32))
    fm8 = jnp.concatenate([fm7, fm8], axis=2)
    fm9 = relu(_conv_layer(ni, vertices, fm8, w9, b9, d9, 32))
    fm9 = jnp.concatenate([fm8, fm9], axis=2)
    vertices, fm9 = _pool(vertices, fm9, ni)

    ni = _ball_query(vertices, 1.0)
    fm10 = relu(_conv_layer(ni, vertices, fm9, w10, b10, d10, 32))
    fm10 = jnp.concatenate([fm9, fm10], axis=2)
    fm11 = _conv_layer(ni, vertices, fm10, w11, b11, d11, 1024)
    return _tail(fm11, cw1, cb1, bng, bnb, cw2, cb2)


# speedup vs baseline: 10.1079x; 2.5736x over previous
import functools

import jax
import jax.numpy as jnp
from jax.experimental import pallas as pl

NSAMPLE = 32


def _hilo(t):
    hi = t.astype(jnp.bfloat16)
    lo = (t - hi.astype(jnp.float32)).astype(jnp.bfloat16)
    return jnp.concatenate([hi, lo], axis=1)


def _oh_dot(oh_bool, table2):
    oh = oh_bool.astype(jnp.bfloat16)
    g2 = jax.lax.dot(oh, table2, preferred_element_type=jnp.float32)
    w = table2.shape[1] // 2
    return g2[:, :w] + g2[:, w:]


def _rec(x):
    hi = x.astype(jnp.bfloat16).astype(jnp.float32)
    lo = (x - hi).astype(jnp.bfloat16).astype(jnp.float32)
    return hi + lo


def _ball_body(pi_ref, pj_ref, o_ref, *, radius, n, nsample):
    pi = pi_ref[0]
    pj = pj_ref[0]
    ssi = jnp.sum(pi * pi, axis=1, keepdims=True)
    ssj = jnp.sum(pj * pj, axis=0, keepdims=True)
    dot = jnp.dot(pi, pj, preferred_element_type=jnp.float32)
    d2 = (-2.0 * dot + ssi) + ssj
    r = pi.shape[0]
    iota = jax.lax.broadcasted_iota(jnp.int32, (r, n), 1)
    cand = jnp.where(d2 <= radius * radius, iota, n)
    cols = []
    first = None
    for _ in range(nsample):
        m = jnp.min(cand, axis=1, keepdims=True)
        if first is None:
            first = m
        cols.append(jnp.where(m == n, first, m))
        cand = jnp.where(cand == m, n, cand)
    o_ref[0] = jnp.concatenate(cols, axis=1)


def _ball_query(vertices, radius):
    b, n, _ = vertices.shape
    rows = min(n, 256)
    vt = jnp.transpose(vertices, (0, 2, 1))
    return pl.pallas_call(
        functools.partial(_ball_body, radius=radius, n=n, nsample=NSAMPLE),
        grid=(b, n // rows),
        in_specs=[pl.BlockSpec((1, rows, 3), lambda bb, ii: (bb, ii, 0)),
                  pl.BlockSpec((1, 3, n), lambda bb, ii: (bb, 0, 0))],
        out_specs=pl.BlockSpec((1, rows, NSAMPLE), lambda bb, ii: (bb, ii, 0)),
        out_shape=jax.ShapeDtypeStruct((b, n, NSAMPLE), jnp.int32),
    )(vertices, vt)


def _mm_body(fm_ref, w_ref, b_ref, o_ref):
    o_ref[0] = (jnp.dot(fm_ref[0], w_ref[...],
                        preferred_element_type=jnp.float32) + b_ref[...])


def _mm(fm, w, bias):
    b, n, ci = fm.shape
    c2 = w.shape[1]
    return pl.pallas_call(
        _mm_body,
        grid=(b,),
        in_specs=[pl.BlockSpec((1, n, ci), lambda bb: (bb, 0, 0)),
                  pl.BlockSpec((ci, c2), lambda bb: (0, 0)),
                  pl.BlockSpec((1, c2), lambda bb: (0, 0))],
        out_specs=pl.BlockSpec((1, n, c2), lambda bb: (bb, 0, 0)),
        out_shape=jax.ShapeDtypeStruct((b, n, c2), jnp.float32),
    )(fm, w, bias.reshape(1, -1))


def _conv_body(ni_ref, xyzb_ref, xyzf_ref, foutb_ref, foutf_ref, sd_ref,
               o_ref, *, co, n, nsample):
    rows = ni_ref.shape[1]
    xyz_b = _rec(xyzb_ref[0])
    xyz_f = xyzf_ref[0]
    fout_f = foutf_ref[0]
    fc = foutb_ref[0][:, :co]
    sd = sd_ref[...]
    sdn = sd / jnp.maximum(
        jnp.sqrt(jnp.sum(sd * sd, axis=0, keepdims=True)), 1e-12)
    cat = _hilo(jnp.concatenate([xyz_f, fout_f[:, co:]], axis=1))
    iota = jax.lax.broadcasted_iota(jnp.int32, (rows, n), 1)
    dns = []
    fss = []
    for k in range(nsample):
        idx = ni_ref[0, :, k:k + 1]
        g = _oh_dot(iota == idx, cat)
        d = g[:, :3] - xyz_b
        nrm = jnp.sqrt(jnp.sum(d * d, axis=1, keepdims=True))
        dns.append(d / jnp.maximum(nrm, 1e-12))
        fss.append(g[:, 3:])
    dn_all = jnp.concatenate(dns, axis=0)
    theta_all = jnp.dot(dn_all, sdn, preferred_element_type=jnp.float32)
    acc = jnp.full((rows, co), -jnp.inf, jnp.float32)
    for k in range(nsample):
        acc = jnp.maximum(acc, theta_all[k * rows:(k + 1) * rows] * fss[k])
    o_ref[0] = fc + acc


def _conv(ni, vertices, fout, sd, co):
    b, n, _ = ni.shape
    rows = min(n, 256)
    c2 = fout.shape[2]
    return pl.pallas_call(
        functools.partial(_conv_body, co=co, n=n, nsample=NSAMPLE),
        grid=(b, n // rows),
        in_specs=[pl.BlockSpec((1, rows, NSAMPLE), lambda bb, ii: (bb, ii, 0)),
                  pl.BlockSpec((1, rows, 3), lambda bb, ii: (bb, ii, 0)),
                  pl.BlockSpec((1, n, 3), lambda bb, ii: (bb, 0, 0)),
                  pl.BlockSpec((1, rows, c2), lambda bb, ii: (bb, ii, 0)),
                  pl.BlockSpec((1, n, c2), lambda bb, ii: (bb, 0, 0)),
                  pl.BlockSpec((3, co), lambda bb, ii: (0, 0))],
        out_specs=pl.BlockSpec((1, rows, co), lambda bb, ii: (bb, ii, 0)),
        out_shape=jax.ShapeDtypeStruct((b, n, co), jnp.float32),
    )(ni, vertices, vertices, fout, fout, sd)


def _conv_layer(ni, vertices, fm, w, bias, directions, co):
    fout = _mm(fm, w, bias)
    return _conv(ni, vertices, fout, directions, co)


def _surf_body(ni_ref, xyzb_ref, xyzf_ref, sd_ref, o_ref, *, co, n, nsample):
    rows = ni_ref.shape[1]
    xyz_b = _rec(xyzb_ref[0])
    xyz_f = xyzf_ref[0]
    sd = sd_ref[...]
    sdn = sd / jnp.maximum(
        jnp.sqrt(jnp.sum(sd * sd, axis=0, keepdims=True)), 1e-12)
    xyz2 = _hilo(xyz_f)
    iota = jax.lax.broadcasted_iota(jnp.int32, (rows, n), 1)
    dns = []
    for k in range(nsample):
        idx = ni_ref[0, :, k:k + 1]
        g = _oh_dot(iota == idx, xyz2)
        d = g - xyz_b
        nrm = jnp.sqrt(jnp.sum(d * d, axis=1, keepdims=True))
        dns.append(d / jnp.maximum(nrm, 1e-12))
    dn_all = jnp.concatenate(dns, axis=0)
    theta_all = jnp.dot(dn_all, sdn, preferred_element_type=jnp.float32)
    acc = jnp.full((rows, co), -jnp.inf, jnp.float32)
    for k in range(nsample):
        acc = jnp.maximum(
            acc, jnp.maximum(theta_all[k * rows:(k + 1) * rows], 0.0))
    o_ref[0] = acc


def _conv_surface(ni, vertices, directions, co):
    b, n, _ = ni.shape
    rows = min(n, 256)
    return pl.pallas_call(
        functools.partial(_surf_body, co=co, n=n, nsample=NSAMPLE),
        grid=(b, n // rows),
        in_specs=[pl.BlockSpec((1, rows, NSAMPLE), lambda bb, ii: (bb, ii, 0)),
                  pl.BlockSpec((1, rows, 3), lambda bb, ii: (bb, ii, 0)),
                  pl.BlockSpec((1, n, 3), lambda bb, ii: (bb, 0, 0)),
                  pl.BlockSpec((3, co), lambda bb, ii: (0, 0))],
        out_specs=pl.BlockSpec((1, rows, co), lambda bb, ii: (bb, ii, 0)),
        out_shape=jax.ShapeDtypeStruct((b, n, co), jnp.float32),
    )(ni, vertices, vertices, directions)


def _pool_body(ni_ref, fmf_ref, o_ref, *, n):
    rows = ni_ref.shape[1]
    fm = _hilo(fmf_ref[0])
    c = fm.shape[1] // 2
    iota = jax.lax.broadcasted_iota(jnp.int32, (rows, n), 1)
    acc = jnp.full((rows, c), -jnp.inf, jnp.float32)
    for k in range(4):
        idx = ni_ref[0, :, k:k + 1]
        acc = jnp.maximum(acc, _oh_dot(iota == idx, fm))
    o_ref[0] = acc


def _pool(vertices, fm, ni):
    b, n, c = fm.shape
    pool_num = n // 2
    rows = min(pool_num, 256)
    ni4 = ni[:, :pool_num, :4]
    pooled = pl.pallas_call(
        functools.partial(_pool_body, n=n),
        grid=(b, pool_num // rows),
        in_specs=[pl.BlockSpec((1, rows, 4), lambda bb, ii: (bb, ii, 0)),
                  pl.BlockSpec((1, n, c), lambda bb, ii: (bb, 0, 0))],
        out_specs=pl.BlockSpec((1, rows, c), lambda bb, ii: (bb, ii, 0)),
        out_shape=jax.ShapeDtypeStruct((b, pool_num, c), jnp.float32),
    )(ni4, fm)
    return vertices[:, :pool_num, :], pooled


def _tail_body(fm_ref, cw1_ref, cb1_ref, bng_ref, bnb_ref, cw2_ref, cb2_ref,
               o_ref):
    m = jnp.max(fm_ref[0], axis=0, keepdims=True)
    x = jnp.dot(m, cw1_ref[...], preferred_element_type=jnp.float32)
    x = x + cb1_ref[...]
    x = bng_ref[...] * x / jnp.sqrt(1.0 + 1e-5) + bnb_ref[...]
    x = jnp.maximum(x, 0.0)
    o_ref[0] = (jnp.dot(x, cw2_ref[...], preferred_element_type=jnp.float32)
                + cb2_ref[...])


def _tail(fm11, cw1, cb1, bng, bnb, cw2, cb2):
    b, n, c = fm11.shape
    h = cw1.shape[1]
    o = cw2.shape[1]
    out = pl.pallas_call(
        _tail_body,
        grid=(b,),
        in_specs=[pl.BlockSpec((1, n, c), lambda bb: (bb, 0, 0)),
                  pl.BlockSpec((c, h), lambda bb: (0, 0)),
                  pl.BlockSpec((1, h), lambda bb: (0, 0)),
                  pl.BlockSpec((1, h), lambda bb: (0, 0)),
                  pl.BlockSpec((1, h), lambda bb: (0, 0)),
                  pl.BlockSpec((h, o), lambda bb: (0, 0)),
                  pl.BlockSpec((1, o), lambda bb: (0, 0))],
        out_specs=pl.BlockSpec((1, 1, o), lambda bb: (bb, 0, 0)),
        out_shape=jax.ShapeDtypeStruct((b, 1, o), jnp.float32),
    )(fm11, cw1, cb1.reshape(1, -1), bng.reshape(1, -1), bnb.reshape(1, -1),
      cw2, cb2.reshape(1, -1))
    return out.reshape(b, o)


def kernel(vertices, dir0, w1, b1, d1, w3, b3, d3, w4, b4, d4, w6, b6, d6,
           w7, b7, d7, w8, b8, d8, w9, b9, d9, w10, b10, d10, w11, b11, d11,
           cw1, cb1, bng, bnb, cw2, cb2):
    relu = jax.nn.relu
    ni = _ball_query(vertices, 0.2)
    fm0 = relu(_conv_surface(ni, vertices, dir0, 32))
    fm1 = relu(_conv_layer(ni, vertices, fm0, w1, b1, d1, 32))
    fm1 = jnp.concatenate([fm0, fm1], axis=2)
    vertices, fm1 = _pool(vertices, fm1, ni)

    ni = _ball_query(vertices, 0.4)
    fm3 = relu(_conv_layer(ni, vertices, fm1, w3, b3, d3, 32))
    fm3 = jnp.concatenate([fm1, fm3], axis=2)
    fm4 = relu(_conv_layer(ni, vertices, fm3, w4, b4, d4, 32))
    fm4 = jnp.concatenate([fm3, fm4], axis=2)
    vertices, fm4 = _pool(vertices, fm4, ni)

    ni = _ball_query(vertices, 0.6)
    fm6 = relu(_conv_layer(ni, vertices, fm4, w6, b6, d6, 32))
    fm6 = jnp.concatenate([fm4, fm6], axis=2)
    fm7 = relu(_conv_layer(ni, vertices, fm6, w7, b7, d7, 32))
    fm7 = jnp.concatenate([fm6, fm7], axis=2)
    vertices, fm7 = _pool(vertices, fm7, ni)

    ni = _ball_query(vertices, 0.8)
    fm8 = relu(_conv_layer(ni, vertices, fm7, w8, b8, d8, 32))
    fm8 = jnp.concatenate([fm7, fm8], axis=2)
    fm9 = relu(_conv_layer(ni, vertices, fm8, w9, b9, d9, 32))
    fm9 = jnp.concatenate([fm8, fm9], axis=2)
    vertices, fm9 = _pool(vertices, fm9, ni)

    ni = _ball_query(vertices, 1.0)
    fm10 = relu(_conv_layer(ni, vertices, fm9, w10, b10, d10, 32))
    fm10 = jnp.concatenate([fm9, fm10], axis=2)
    fm11 = _conv_layer(ni, vertices, fm10, w11, b11, d11, 1024)
    return _tail(fm11, cw1, cb1, bng, bnb, cw2, cb2)
